# Initial kernel scaffold; baseline (speedup 1.0000x reference)
import jax
import jax.numpy as jnp
from jax.experimental import pallas as pl


def kernel(tokens, table):
    raise NotImplementedError("write your pallas kernel here")


# speedup vs baseline: 5.0806x; 5.0806x over previous
import functools
import math

import jax
import jax.numpy as jnp
from jax import lax
from jax.experimental import pallas as pl
from jax.experimental.pallas import tpu as pltpu
from jax.experimental.pallas import tpu_sc as plsc

VOCAB = 100000
EMBED = 128
SCALE = math.sqrt(EMBED)

ROWS = 4096 * 200
NC, NS = 2, 16
NW = NC * NS
RW = ROWS // NW
CH = 128
NCHUNK = RW // CH

_mesh = plsc.VectorSubcoreMesh(core_axis_name="c", subcore_axis_name="s")


@functools.partial(
    pl.kernel,
    mesh=_mesh,
    out_type=jax.ShapeDtypeStruct((ROWS, EMBED), jnp.float32),
    scratch_types=[
        pltpu.VMEM((NCHUNK, CH), jnp.int32),
        pltpu.VMEM((CH, EMBED), jnp.float32),
        pltpu.SemaphoreType.DMA,
    ],
)
def _embed_sc(tok_hbm, tab_hbm, out_hbm, idx_v, rows_v, sem):
    wid = lax.axis_index("s") * NC + lax.axis_index("c")
    base = wid * RW
    pltpu.sync_copy(tok_hbm.at[wid], idx_v)

    def chunk(j, carry):
        pltpu.async_copy(tab_hbm.at[idx_v.at[j]], rows_v, sem).wait()

        def row(r, c2):
            for c in range(EMBED // 16):
                rows_v[r, pl.ds(c * 16, 16)] = rows_v[r, pl.ds(c * 16, 16)] * SCALE
            return c2

        lax.fori_loop(0, CH, row, 0)
        pltpu.sync_copy(rows_v, out_hbm.at[pl.ds(base + j * CH, CH)])
        return carry

    lax.fori_loop(0, NCHUNK, chunk, 0)


def kernel(tokens, table):
    tok = tokens.reshape(NW, NCHUNK, CH).astype(jnp.int32)
    out = _embed_sc(tok, table)
    return out.reshape(tokens.shape[0], tokens.shape[1], EMBED)


# speedup vs baseline: 8.6509x; 1.7027x over previous
import functools
import math

import jax
import jax.numpy as jnp
from jax import lax
from jax.experimental import pallas as pl
from jax.experimental.pallas import tpu as pltpu
from jax.experimental.pallas import tpu_sc as plsc

VOCAB = 100000
EMBED = 128
SCALE = math.sqrt(EMBED)

ROWS = 4096 * 200
NC, NS = 2, 16
NW = NC * NS
RW = ROWS // NW
CH = 128
NCHUNK = RW // CH

_mesh = plsc.VectorSubcoreMesh(core_axis_name="c", subcore_axis_name="s")


@functools.partial(
    pl.kernel,
    mesh=_mesh,
    out_type=jax.ShapeDtypeStruct((ROWS, EMBED), jnp.float32),
    scratch_types=[
        pltpu.VMEM((NCHUNK, CH), jnp.int32),
        pltpu.VMEM((CH, EMBED), jnp.float32),
        pltpu.VMEM((CH, EMBED), jnp.float32),
        pltpu.SemaphoreType.DMA,
        pltpu.SemaphoreType.DMA,
        pltpu.SemaphoreType.DMA,
        pltpu.SemaphoreType.DMA,
    ],
)
def _embed_sc(tok_hbm, tab_hbm, out_hbm, idx_v, rows0, rows1,
              g0, g1, o0, o1):
    wid = lax.axis_index("s") * NC + lax.axis_index("c")
    base = wid * RW
    rows = (rows0, rows1)
    gsem = (g0, g1)
    osem = (o0, o1)

    pltpu.sync_copy(tok_hbm.at[wid], idx_v)

    def g_start(c, b):
        pltpu.async_copy(tab_hbm.at[idx_v.at[c]], rows[b], gsem[b])

    def g_wait(c, b):
        pltpu.make_async_copy(tab_hbm.at[idx_v.at[c]], rows[b], gsem[b]).wait()

    def o_start(c, b):
        pltpu.async_copy(rows[b], out_hbm.at[pl.ds(base + c * CH, CH)], osem[b])

    def o_wait(c, b):
        pltpu.make_async_copy(
            rows[b], out_hbm.at[pl.ds(base + c * CH, CH)], osem[b]).wait()

    g_start(0, 0)

    def outer(i, carry):
        j = i * 2
        for b in range(2):
            cur = j + b
            other = 1 - b

            @pl.when(cur > 0)
            def _():
                o_wait(cur - 1, other)

            @pl.when(cur + 1 < NCHUNK)
            def _():
                g_start(cur + 1, other)

            g_wait(cur, b)

            buf = rows[b]

            def row(r, c2):
                for c in range(EMBED // 16):
                    buf[r, pl.ds(c * 16, 16)] = buf[r, pl.ds(c * 16, 16)] * SCALE
                return c2

            lax.fori_loop(0, CH, row, 0)
            o_start(cur, b)
        return carry

    lax.fori_loop(0, NCHUNK // 2, outer, 0)
    o_wait(NCHUNK - 1, (NCHUNK - 1) % 2)


def kernel(tokens, table):
    tok = tokens.reshape(NW, NCHUNK, CH).astype(jnp.int32)
    out = _embed_sc(tok, table)
    return out.reshape(tokens.shape[0], tokens.shape[1], EMBED)
